# Initial kernel scaffold; baseline (speedup 1.0000x reference)
import jax
import jax.numpy as jnp
from jax.experimental import pallas as pl


def kernel(node_feat, edge_feat, labels, edge_index, node_graph_ids, W_n2l, b_n2l, W_e2l, W_conv, b_conv, W_out, b_out, W_h1, b_h1, W_h2, b_h2):
    raise NotImplementedError("write your pallas kernel here")


# speedup vs baseline: 2.8685x; 2.8685x over previous
import functools

import jax
import jax.numpy as jnp
from jax import lax
from jax.experimental import pallas as pl
from jax.experimental.pallas import tpu as pltpu
from jax.experimental.pallas import tpu_sc as plsc

N = 10000
E = 320000
NF = 128
EF = 16
LATENT = 128
HF = LATENT // 2
OUT = 128
HID = 256
G = 128
MAX_LV = 3

NC = 2
NS = 16
NW = NC * NS
CHUNK = 96
NCH = 106
EPW = NCH * CHUNK
E_PAD = NW * EPW
NP = 10112
RPT = NP // NS
DUMMY = N + 8
BN = 632
NB = NP // BN

_mesh = plsc.VectorSubcoreMesh(
    core_axis_name="c", subcore_axis_name="s", num_cores=NC, num_subcores=NS)


def _zero_acc(zbuf, acc, s, width):
    zvec = jnp.zeros((16,), jnp.float32)
    for r in range(8):
        for q in range(width // 16):
            zbuf[r, pl.ds(q * 16, 16)] = zvec
    base = s * RPT

    def zloop(t, carry):
        pltpu.sync_copy(zbuf, acc.at[pl.ds(base + t * 8, 8)])
        return carry

    lax.fori_loop(0, RPT // 8, zloop, None)


def _scatter_pass(data_hbm, idx, didx, rows, sems, acc, gather):
    def fetch(j, b):
        if gather:
            return pltpu.async_copy(data_hbm.at[idx.at[j]], rows[b], sems[b])
        return pltpu.async_copy(data_hbm.at[j], rows[b], sems[b])

    def fetch_wait(j, b):
        if gather:
            pltpu.make_async_copy(data_hbm.at[idx.at[j]], rows[b],
                                  sems[b]).wait()
        else:
            pltpu.make_async_copy(data_hbm.at[j], rows[b], sems[b]).wait()

    fetch(0, 0)

    def outer(jj, carry):
        for b in range(2):
            j = jj * 2 + b
            fetch_wait(j, b)
            nxt = j + 1

            @pl.when(nxt < NCH)
            def _():
                fetch(nxt, 1 - b)

            pltpu.sync_copy(rows[b], acc.at[didx.at[j]], add=True)
        return carry

    lax.fori_loop(0, NCH // 2, outer, None)


def _copy_out(acc, out_ref, s):
    pltpu.sync_copy(acc.at[pl.ds(s * RPT, RPT)],
                    out_ref.at[pl.ds(s * RPT, RPT)])


@functools.partial(
    pl.kernel,
    out_type=jax.ShapeDtypeStruct((NC, NP, EF), jnp.float32),
    mesh=_mesh,
    compiler_params=pltpu.CompilerParams(use_tc_tiling_on_sc=False),
    scratch_types=[
        pltpu.VMEM((NCH, CHUNK), jnp.int32),
        pltpu.VMEM((CHUNK, EF), jnp.float32),
        pltpu.VMEM((CHUNK, EF), jnp.float32),
        pltpu.VMEM((8, EF), jnp.float32),
        pltpu.VMEM_SHARED((NP, EF), jnp.float32),
        pltpu.SemaphoreType.DMA,
        pltpu.SemaphoreType.DMA,
    ],
)
def _sc_e2n(ef_hbm, dst_hbm, out_hbm, didx, rows0, rows1, zbuf, acc,
            sem0, sem1):
    c = lax.axis_index("c")
    s = lax.axis_index("s")
    wid = c * NS + s
    pltpu.sync_copy(dst_hbm.at[wid], didx)
    _zero_acc(zbuf, acc, s, EF)
    plsc.subcore_barrier()
    _scatter_pass(ef_hbm.at[wid], None, didx, (rows0, rows1), (sem0, sem1),
                  acc, gather=False)
    plsc.subcore_barrier()
    _copy_out(acc, out_hbm.at[c], s)


@functools.partial(
    pl.kernel,
    out_type=jax.ShapeDtypeStruct((NC, 2, NP, HF), jnp.float32),
    mesh=_mesh,
    compiler_params=pltpu.CompilerParams(use_tc_tiling_on_sc=False),
    scratch_types=[
        pltpu.VMEM((NCH, CHUNK), jnp.int32),
        pltpu.VMEM((NCH, CHUNK), jnp.int32),
        pltpu.VMEM((CHUNK, HF), jnp.float32),
        pltpu.VMEM((CHUNK, HF), jnp.float32),
        pltpu.VMEM((8, HF), jnp.float32),
        pltpu.VMEM_SHARED((NP, HF), jnp.float32),
        pltpu.SemaphoreType.DMA,
        pltpu.SemaphoreType.DMA,
    ],
)
def _sc_n2n(cur0_hbm, cur1_hbm, src_hbm, dst_hbm, out_hbm, sidx, didx,
            rows0, rows1, zbuf, acc, sem0, sem1):
    c = lax.axis_index("c")
    s = lax.axis_index("s")
    wid = c * NS + s
    pltpu.sync_copy(src_hbm.at[wid], sidx)
    pltpu.sync_copy(dst_hbm.at[wid], didx)
    for h, cur_hbm in enumerate((cur0_hbm, cur1_hbm)):
        _zero_acc(zbuf, acc, s, HF)
        plsc.subcore_barrier()
        _scatter_pass(cur_hbm, sidx, didx, (rows0, rows1), (sem0, sem1),
                      acc, gather=True)
        plsc.subcore_barrier()
        _copy_out(acc, out_hbm.at[c, h], s)
        if h == 0:
            plsc.subcore_barrier()


def _tc_msg_body(nf, e0, e1, wn, bn, we, out0, out1):
    pool = e0[...] + e1[...]
    x = jnp.dot(nf[...], wn[...], preferred_element_type=jnp.float32)
    x = x + jnp.dot(pool, we[...], preferred_element_type=jnp.float32)
    x = jnp.maximum(x + bn[...], 0.0)
    out0[...] = x[:, :HF]
    out1[...] = x[:, HF:]


def _tc_lvl_body(a00, a01, a10, a11, wc0, wc1, bc, msg0, msg1, out0, out1):
    x = jnp.dot(a00[...] + a10[...], wc0[...],
                preferred_element_type=jnp.float32)
    x = x + jnp.dot(a01[...] + a11[...], wc1[...],
                    preferred_element_type=jnp.float32)
    x = x + bc[...]
    out0[...] = jnp.maximum(x[:, :HF] + msg0[...], 0.0)
    out1[...] = jnp.maximum(x[:, HF:] + msg1[...], 0.0)


def _tc_head_body(ids, cur0, cur1, labels, wo, bo, w1, b1, w2, b2,
                  pred_out, mae_out, mse_out, yacc):
    i = pl.program_id(0)

    @pl.when(i == 0)
    def _():
        yacc[...] = jnp.zeros((G, LATENT), jnp.float32)

    cur = jnp.concatenate([cur0[...], cur1[...]], axis=1)
    onehot = (lax.broadcasted_iota(jnp.int32, (G, BN), 0)
              == ids[0]).astype(jnp.float32)
    yacc[...] += jnp.dot(onehot, cur, preferred_element_type=jnp.float32)

    @pl.when(i == NB - 1)
    def _():
        y = yacc[...]
        embed = jnp.maximum(
            jnp.dot(y, wo[...], preferred_element_type=jnp.float32) + bo[...],
            0.0)
        h1 = jnp.maximum(
            jnp.dot(embed, w1[...], preferred_element_type=jnp.float32)
            + b1[...], 0.0)
        pred = jnp.sum(h1 * w2[...], axis=1, keepdims=True) + b2[0, 0]
        pred_out[...] = pred
        d = pred - labels[...]
        mse_out[...] = jnp.mean(d * d, keepdims=True)
        mae_out[...] = jnp.mean(jnp.abs(d), keepdims=True)


def _const2(shape):
    return pl.BlockSpec(shape, lambda i: (0, 0))


def kernel(node_feat, edge_feat, labels, edge_index, node_graph_ids,
           W_n2l, b_n2l, W_e2l, W_conv, b_conv, W_out, b_out,
           W_h1, b_h1, W_h2, b_h2):
    f32 = jnp.float32
    src = edge_index[0]
    dst = edge_index[1]
    pad_e = E_PAD - E
    src3 = jnp.concatenate(
        [src, jnp.zeros((pad_e,), jnp.int32)]).reshape(NW, NCH, CHUNK)
    dst3 = jnp.concatenate(
        [dst, jnp.full((pad_e,), DUMMY, jnp.int32)]).reshape(NW, NCH, CHUNK)
    ef4 = jnp.concatenate(
        [edge_feat, jnp.zeros((pad_e, EF), f32)]).reshape(NW, NCH, CHUNK, EF)
    nfp = jnp.zeros((NP, NF), f32).at[:N].set(node_feat)
    ids3 = jnp.full((NP,), G, jnp.int32).at[:N].set(
        node_graph_ids).reshape(NB, 1, BN)

    e2n = _sc_e2n(ef4, dst3)

    half_spec = pl.BlockSpec((BN, HF), lambda i: (i, 0))
    msg0, msg1 = pl.pallas_call(
        _tc_msg_body,
        grid=(NB,),
        in_specs=[
            pl.BlockSpec((BN, NF), lambda i: (i, 0)),
            pl.BlockSpec((BN, EF), lambda i: (i, 0)),
            pl.BlockSpec((BN, EF), lambda i: (i, 0)),
            _const2((NF, LATENT)),
            _const2((1, LATENT)),
            _const2((EF, LATENT)),
        ],
        out_specs=[half_spec, half_spec],
        out_shape=[jax.ShapeDtypeStruct((NP, HF), f32)] * 2,
    )(nfp, e2n[0], e2n[1], W_n2l, b_n2l.reshape(1, LATENT), W_e2l)

    Wc0 = W_conv[:HF]
    Wc1 = W_conv[HF:]

    def _level(_, carry):
        cur0, cur1 = carry
        acc = _sc_n2n(cur0, cur1, src3, dst3)
        return pl.pallas_call(
            _tc_lvl_body,
            grid=(NB,),
            in_specs=[half_spec, half_spec, half_spec, half_spec,
                      _const2((HF, LATENT)), _const2((HF, LATENT)),
                      _const2((1, LATENT)), half_spec, half_spec],
            out_specs=[half_spec, half_spec],
            out_shape=[jax.ShapeDtypeStruct((NP, HF), f32)] * 2,
        )(acc[0, 0], acc[0, 1], acc[1, 0], acc[1, 1], Wc0, Wc1,
          b_conv.reshape(1, LATENT), msg0, msg1)

    cur0, cur1 = lax.fori_loop(0, MAX_LV, _level, (msg0, msg1))

    pred, mae, mse = pl.pallas_call(
        _tc_head_body,
        grid=(NB,),
        in_specs=[
            pl.BlockSpec((1, 1, BN), lambda i: (i, 0, 0)),
            half_spec,
            half_spec,
            _const2((G, 1)),
            _const2((LATENT, OUT)),
            _const2((1, OUT)),
            _const2((OUT, HID)),
            _const2((1, HID)),
            _const2((1, HID)),
            _const2((1, 1)),
        ],
        out_specs=[_const2((G, 1)), _const2((1, 1)), _const2((1, 1))],
        out_shape=[
            jax.ShapeDtypeStruct((G, 1), f32),
            jax.ShapeDtypeStruct((1, 1), f32),
            jax.ShapeDtypeStruct((1, 1), f32),
        ],
        scratch_shapes=[pltpu.VMEM((G, LATENT), f32)],
    )(ids3, cur0, cur1, labels, W_out, b_out.reshape(1, OUT), W_h1,
      b_h1.reshape(1, HID), W_h2.reshape(1, HID), b_h2.reshape(1, 1))

    return pred, mae.reshape(()), mse.reshape(())
